# baseline (speedup 1.0000x reference)
import functools

import jax
import jax.numpy as jnp
from jax import lax
from jax.experimental import pallas as pl
from jax.experimental.pallas import tpu as pltpu
from jax.experimental.pallas import tpu_sc as plsc

BATCH = 4096
SEQ = 200
D = 64
NW = 32
BB = BATCH // NW
L = 16

_mesh = plsc.VectorSubcoreMesh(core_axis_name="c", subcore_axis_name="s")


@functools.partial(
    pl.kernel,
    mesh=_mesh,
    out_type=jax.ShapeDtypeStruct((SEQ, 8, NW, 8, 128), jnp.float32),
    scratch_types=[
        pltpu.VMEM((8, 128), jnp.int32),
        [pltpu.VMEM((128,), jnp.int32)] * 2,
        [pltpu.VMEM((128,), jnp.int32)] * 2,
        [pltpu.VMEM((128, 128), jnp.float32)] * 2,
        [pltpu.VMEM((64, 128), jnp.float32)] * 2,
        pltpu.VMEM((8, D), jnp.float32),
        [pltpu.SemaphoreType.DMA] * 2,
        [pltpu.SemaphoreType.DMA] * 2,
    ],
    compiler_params=pltpu.CompilerParams(
        use_tc_tiling_on_sc=True, needs_layout_passes=False,
        disable_bounds_checks=True),
)
def _emb_kernel(xt_hbm, pos_hbm, tab2_hbm, out_hbm,
                idxt_v, idx2_v, base_v, rows_v, slab_v, posb_v, gsems, ssems):
    wid = lax.axis_index("s") * 2 + lax.axis_index("c")
    iota = lax.iota(jnp.int32, L)

    def build_idx(sj, b):
        for g in range(8):
            sl = pl.ds(g * L, L)
            tv = idxt_v[sj, sl]
            idx2_v[b][sl] = lax.shift_right_logical(tv, 1)
            base_v[b][sl] = (tv & 1) * 64

    def s8_body(s8, _):
        pltpu.sync_copy(xt_hbm.at[pl.ds(s8 * 8, 8), pl.ds(wid * 128, 128)],
                        idxt_v)
        pltpu.sync_copy(pos_hbm.at[0, pl.ds(s8 * 8, 8)], posb_v)

        build_idx(0, 0)
        pltpu.async_copy(tab2_hbm.at[idx2_v[0]], rows_v[0], gsems[0])
        for sj in range(8):
            b = sj % 2
            if sj < 7:
                nb = 1 - b
                build_idx(sj + 1, nb)
                pltpu.async_copy(
                    tab2_hbm.at[idx2_v[nb]], rows_v[nb], gsems[nb])
            pltpu.make_async_copy(
                tab2_hbm.at[idx2_v[b]], rows_v[b], gsems[b]).wait()
            if sj >= 2:
                for ci in range(8):
                    pltpu.make_async_copy(
                        slab_v[b].at[pl.ds(ci * 8, 8)],
                        out_hbm.at[s8 * 8 + sj - 2, ci, wid],
                        ssems[b]).wait()
            sjv = jnp.int32(sj)

            def c_body(c4, _):
                for u in range(4):
                    c = c4 * 4 + u
                    pvec = plsc.load_gather(
                        posb_v, [jnp.broadcast_to(sjv, (L,)),
                                 jnp.broadcast_to(c, (L,))])
                    for g in range(8):
                        sl = pl.ds(g * L, L)
                        rg = iota + g * L
                        cg = base_v[b][sl] + c
                        val = plsc.load_gather(rows_v[b], [rg, cg]) + pvec
                        slab_v[b][c, sl] = val
                return 0

            lax.fori_loop(0, D // 4, c_body, 0)
            for ci in range(8):
                pltpu.async_copy(slab_v[b].at[pl.ds(ci * 8, 8)],
                                 out_hbm.at[s8 * 8 + sj, ci, wid], ssems[b])
        for sj in (6, 7):
            b = sj % 2
            for ci in range(8):
                pltpu.make_async_copy(
                    slab_v[b].at[pl.ds(ci * 8, 8)],
                    out_hbm.at[s8 * 8 + sj, ci, wid], ssems[b]).wait()
        return 0

    lax.fori_loop(0, SEQ // 8, s8_body, 0)


def kernel(x, token_emb_table, pos_emb):
    xt = jnp.transpose(x)
    tab2 = token_emb_table.reshape(500000, 128)
    out5 = _emb_kernel(xt, pos_emb, tab2)
    return out5.transpose(2, 4, 0, 1, 3).reshape(BATCH, SEQ, D)


# speedup vs baseline: 2.2030x; 2.2030x over previous
import functools

import jax
import jax.numpy as jnp
from jax import lax
from jax.experimental import pallas as pl
from jax.experimental.pallas import tpu as pltpu
from jax.experimental.pallas import tpu_sc as plsc

BATCH = 4096
SEQ = 200
D = 64
NW = 32
PER_W = BATCH // NW
LANES = 16
NBUF = 4
ROWS_PER_STEP = 4

_mesh = plsc.VectorSubcoreMesh(core_axis_name="c", subcore_axis_name="s")


@functools.partial(
    pl.kernel,
    mesh=_mesh,
    out_type=jax.ShapeDtypeStruct((BATCH, SEQ, D), jnp.float32),
    scratch_types=[
        pltpu.VMEM((PER_W, 128), jnp.int32),
        pltpu.VMEM((PER_W, 72), jnp.int32),
        pltpu.VMEM((NBUF, SEQ, D), jnp.float32),
        pltpu.VMEM((1, SEQ, D), jnp.float32),
        [pltpu.SemaphoreType.DMA] * NBUF,
        [pltpu.SemaphoreType.DMA] * NBUF,
    ],
    compiler_params=pltpu.CompilerParams(
        use_tc_tiling_on_sc=False, skip_device_barrier=True),
)
def _emb_kernel(idx_hbm, pos_hbm, table_hbm, out_hbm,
                idx_lo, idx_hi, rows_v, pos_v, gsems, ssems):
    wid = lax.axis_index("s") * 2 + lax.axis_index("c")
    base = wid * PER_W
    HL, HH = 128, 72
    pltpu.sync_copy(pos_hbm, pos_v)
    pltpu.sync_copy(idx_hbm.at[pl.ds(base, PER_W), pl.ds(0, HL)], idx_lo)
    pltpu.sync_copy(idx_hbm.at[pl.ds(base, PER_W), pl.ds(HL, HH)], idx_hi)

    def start_gather(k, b):
        pltpu.async_copy(
            table_hbm.at[idx_lo.at[k]], rows_v.at[b, pl.ds(0, HL)], gsems[b])
        pltpu.async_copy(
            table_hbm.at[idx_hi.at[k]], rows_v.at[b, pl.ds(HL, HH)], gsems[b])

    def wait_gather(k, b):
        pltpu.make_async_copy(
            table_hbm.at[idx_lo.at[k]], rows_v.at[b, pl.ds(0, HL)],
            gsems[b]).wait()
        pltpu.make_async_copy(
            table_hbm.at[idx_hi.at[k]], rows_v.at[b, pl.ds(HL, HH)],
            gsems[b]).wait()

    def start_store(k, b):
        pltpu.async_copy(rows_v.at[b], out_hbm.at[base + k], ssems[b])

    def wait_store(k, b):
        pltpu.make_async_copy(
            rows_v.at[b], out_hbm.at[base + k], ssems[b]).wait()

    def add_pos(b):
        def add_rows(i, _):
            for r in range(ROWS_PER_STEP):
                for j in range(D // LANES):
                    sl = pl.ds(j * LANES, LANES)
                    row = i * ROWS_PER_STEP + r
                    rows_v[b, row, sl] = rows_v[b, row, sl] + pos_v[0, row, sl]
            return 0

        lax.fori_loop(0, SEQ // ROWS_PER_STEP, add_rows, 0)

    def stage(k, b, issue_j):
        wait_gather(k, b)
        add_pos(b)
        start_store(k, b)
        if issue_j:
            j = k + NBUF - 1
            bj = (b + NBUF - 1) % NBUF
            wait_store(j - NBUF, bj)
            start_gather(j, bj)

    for b in range(NBUF - 1):
        start_gather(b, b)

    wait_gather(0, 0)
    add_pos(0)
    start_store(0, 0)
    start_gather(NBUF - 1, NBUF - 1)
    for b in range(1, NBUF):
        stage(b, b, issue_j=True)

    def outer(k4, _):
        k0 = k4 * NBUF
        for b in range(NBUF):
            stage(k0 + b, b, issue_j=True)
        return 0

    lax.fori_loop(1, PER_W // NBUF - 1, outer, 0)

    kl = PER_W - NBUF
    stage(kl, 0, issue_j=True)
    for b in range(1, NBUF):
        stage(kl + b, b, issue_j=False)

    for b in range(NBUF):
        wait_store(kl + b, b)


def kernel(x, token_emb_table, pos_emb):
    return _emb_kernel(x, pos_emb, token_emb_table)
